# Initial kernel scaffold; baseline (speedup 1.0000x reference)
import jax
import jax.numpy as jnp
from jax.experimental import pallas as pl


def kernel(x, embed_matrix):
    raise NotImplementedError("write your pallas kernel here")


# speedup vs baseline: 1.8713x; 1.8713x over previous
import functools

import jax
import jax.numpy as jnp
from jax import lax
from jax.experimental import pallas as pl
from jax.experimental.pallas import tpu as pltpu
from jax.experimental.pallas import tpu_sc as plsc

NW = 32
SUB = 128
K = 4
CHUNK = K * SUB
NBUF = 2


@functools.partial(jax.jit, static_argnums=(2, 3))
def _embedding_gather(idx, table, total, d):
    per_w = total // NW
    g_count = per_w // CHUNK
    mesh = plsc.VectorSubcoreMesh(core_axis_name="c", subcore_axis_name="s")

    @functools.partial(
        pl.kernel,
        mesh=mesh,
        out_type=jax.ShapeDtypeStruct((total, d), jnp.float32),
        compiler_params=pltpu.CompilerParams(use_tc_tiling_on_sc=False),
        scratch_types=[
            pltpu.VMEM((NBUF, K, SUB), jnp.int32),
            pltpu.VMEM((NBUF, CHUNK, d), jnp.float32),
            pltpu.SemaphoreType.DMA,
            pltpu.SemaphoreType.DMA,
            pltpu.SemaphoreType.DMA,
        ],
    )
    def body(idx_hbm, tab_hbm, out_hbm, idx_v, rows_v, sem_i, sem_g, sem_o):
        wid = lax.axis_index("s") * 2 + lax.axis_index("c")
        base = wid * per_w

        for b in range(NBUF):
            pltpu.async_copy(idx_hbm.at[wid, b], idx_v.at[b], sem_i)

        def step(gg, carry):
            for b in range(NBUF):
                g = gg * NBUF + b
                pltpu.make_async_copy(idx_hbm.at[wid, b], idx_v.at[b], sem_i).wait()
                @pl.when(g >= NBUF)
                def _():
                    pltpu.make_async_copy(
                        rows_v.at[b], out_hbm.at[pl.ds(base, CHUNK)], sem_o
                    ).wait()

                handles = [
                    pltpu.async_copy(
                        tab_hbm.at[idx_v.at[b, j]],
                        rows_v.at[b, pl.ds(j * SUB, SUB)],
                        sem_g,
                    )
                    for j in range(K)
                ]
                for h in handles:
                    h.wait()
                pltpu.async_copy(
                    rows_v.at[b], out_hbm.at[pl.ds(base + g * CHUNK, CHUNK)], sem_o
                )
                @pl.when(g + NBUF < g_count)
                def _():
                    pltpu.async_copy(
                        idx_hbm.at[wid, (gg + 1) * NBUF + b], idx_v.at[b], sem_i
                    )
            return carry

        lax.fori_loop(0, g_count // NBUF, step, 0)
        for b in range(NBUF):
            pltpu.make_async_copy(
                rows_v.at[b], out_hbm.at[pl.ds(base, CHUNK)], sem_o
            ).wait()

    return body(idx, table)


def kernel(x, embed_matrix):
    bsz, hist = x.shape
    v, d = embed_matrix.shape
    total = bsz * hist
    per_w = total // NW
    idx = x.reshape(NW, per_w // CHUNK, K, SUB).astype(jnp.int32)
    out = _embedding_gather(idx, embed_matrix, total, d)
    return out.reshape(bsz, hist, d)
